# Initial kernel scaffold; baseline (speedup 1.0000x reference)
import jax
import jax.numpy as jnp
from jax.experimental import pallas as pl


def kernel(x, edge_index, edge_type, batch, rel_w, root_w, root_b, mlp_w1, mlp_b1, mlp_w2, mlp_b2, mlp_w3, mlp_b3):
    raise NotImplementedError("write your pallas kernel here")


# speedup vs baseline: 13.0751x; 13.0751x over previous
import functools

import jax
import jax.numpy as jnp
from jax import lax
from jax.experimental import pallas as pl
from jax.experimental.pallas import tpu as pltpu
from jax.experimental.pallas import tpu_sc as plsc

N = 10000
E = 320000
D = 128
R = 8
G = 64
C = 4

NC = 2
NS = 16
NW = NC * NS

CH = 80
EPW = E // NW
NCHUNK = EPW // CH
CNT_PAD = 81920
CNT_PT = CNT_PAD // NS
CNT_FB = 1280
N_PAD = 10240
NPT = N_PAD // NS
AFB = 128


def _mesh():
    return plsc.VectorSubcoreMesh(core_axis_name="c", subcore_axis_name="s")


def _count(seg):
    @functools.partial(
        pl.kernel,
        compiler_params=pltpu.CompilerParams(needs_layout_passes=False),
        out_type=jax.ShapeDtypeStruct((CNT_PAD,), jnp.float32),
        mesh=_mesh(),
        scratch_types=[
            pltpu.VMEM((CH,), jnp.int32),
            pltpu.VMEM((CH,), jnp.float32),
            pltpu.VMEM((CNT_FB,), jnp.float32),
            pltpu.VMEM_SHARED((CNT_PAD,), jnp.float32),
        ],
    )
    def k(seg_hbm, out_hbm, seg_v, ones_v, fb, cnt_sh):
        c = lax.axis_index("c")
        s = lax.axis_index("s")
        for g in range(CH // 16):
            ones_v[pl.ds(g * 16, 16)] = jnp.ones((16,), jnp.float32)

        def zb(t, cy):
            fb[pl.ds(t * 16, 16)] = jnp.zeros((16,), jnp.float32)
            return cy

        lax.fori_loop(0, CNT_FB // 16, zb, 0)

        @pl.when(c == 0)
        def _():
            for j in range(CNT_PT // CNT_FB):
                pltpu.sync_copy(fb, cnt_sh.at[pl.ds(s * CNT_PT + j * CNT_FB, CNT_FB)])

        plsc.subcore_barrier()

        @pl.when(c == 0)
        def _():
            base = s * (E // NS)

            def chunk(j, cy):
                pltpu.sync_copy(seg_hbm.at[pl.ds(base + j * CH, CH)], seg_v)
                pltpu.sync_copy(ones_v, cnt_sh.at[seg_v], add=True)
                return cy

            lax.fori_loop(0, E // NS // CH, chunk, 0)

        plsc.subcore_barrier()

        @pl.when(c == 0)
        def _():
            def fl(j, cy):
                off = s * CNT_PT + j * CNT_FB
                pltpu.sync_copy(cnt_sh.at[pl.ds(off, CNT_FB)], fb)
                pltpu.sync_copy(fb, out_hbm.at[pl.ds(off, CNT_FB)])
                return cy

            lax.fori_loop(0, CNT_PT // CNT_FB, fl, 0)

    return k(seg)


def _weights(cnt, seg):
    @functools.partial(
        pl.kernel,
        compiler_params=pltpu.CompilerParams(needs_layout_passes=False),
        out_type=jax.ShapeDtypeStruct((E,), jnp.float32),
        mesh=_mesh(),
        scratch_types=[
            pltpu.VMEM((N * R,), jnp.float32),
            pltpu.VMEM((CH,), jnp.int32),
            pltpu.VMEM((CH,), jnp.float32),
        ],
    )
    def k(cnt_hbm, seg_hbm, w_hbm, cnt_v, seg_v, w_v):
        c = lax.axis_index("c")
        s = lax.axis_index("s")
        wid = s * NC + c
        base = wid * EPW
        pltpu.sync_copy(cnt_hbm.at[pl.ds(0, N * R)], cnt_v)

        def chunk(j, cy):
            off = base + j * CH
            pltpu.sync_copy(seg_hbm.at[pl.ds(off, CH)], seg_v)
            for g in range(CH // 16):
                s16 = seg_v[pl.ds(g * 16, 16)]
                c16 = plsc.load_gather(cnt_v, [s16])
                w_v[pl.ds(g * 16, 16)] = 1.0 / jnp.maximum(c16, 1.0)
            pltpu.sync_copy(w_v, w_hbm.at[pl.ds(off, CH)])
            return cy

        lax.fori_loop(0, NCHUNK, chunk, 0)

    return k(cnt, seg)


def _agg(y, gidx, dst, w):
    @functools.partial(
        pl.kernel,
        compiler_params=pltpu.CompilerParams(needs_layout_passes=False),
        out_type=jax.ShapeDtypeStruct((NC * N_PAD, D), jnp.float32),
        mesh=_mesh(),
        scratch_types=[
            pltpu.VMEM((CH,), jnp.int32),
            pltpu.VMEM((CH,), jnp.int32),
            pltpu.VMEM((CH,), jnp.float32),
            pltpu.VMEM((CH, D), jnp.float32),
            pltpu.VMEM((AFB, D), jnp.float32),
            pltpu.VMEM_SHARED((N_PAD, D), jnp.float32),
            pltpu.SemaphoreType.DMA,
        ],
    )
    def k(y_hbm, gi_hbm, d_hbm, w_hbm, out_hbm, gi_v, d_v, w_v, rows_v, fbuf,
          acc_sh, sem):
        c = lax.axis_index("c")
        s = lax.axis_index("s")
        wid = s * NC + c

        def zb(t, cy):
            fbuf[t // 8, pl.ds((t % 8) * 16, 16)] = jnp.zeros((16,), jnp.float32)
            return cy

        lax.fori_loop(0, AFB * 8, zb, 0)
        for j in range(NPT // AFB):
            pltpu.sync_copy(fbuf, acc_sh.at[pl.ds(s * NPT + j * AFB, AFB)])
        plsc.subcore_barrier()

        base = wid * EPW

        def chunk(j, cy):
            off = base + j * CH
            pltpu.sync_copy(gi_hbm.at[pl.ds(off, CH)], gi_v)
            pltpu.sync_copy(d_hbm.at[pl.ds(off, CH)], d_v)
            pltpu.sync_copy(w_hbm.at[pl.ds(off, CH)], w_v)
            pltpu.async_copy(y_hbm.at[gi_v], rows_v, sem).wait()

            def edge(e, cy2):
                wb = plsc.load_gather(w_v, [jnp.zeros((16,), jnp.int32) + e])
                for kk in range(D // 16):
                    sl = pl.ds(kk * 16, 16)
                    rows_v[e, sl] = rows_v[e, sl] * wb
                return cy2

            lax.fori_loop(0, CH, edge, 0)
            pltpu.sync_copy(rows_v, acc_sh.at[d_v], add=True)
            return cy

        lax.fori_loop(0, NCHUNK, chunk, 0)
        plsc.subcore_barrier()

        def fl(j, cy):
            off = s * NPT + j * AFB
            pltpu.sync_copy(acc_sh.at[pl.ds(off, AFB)], fbuf)
            pltpu.sync_copy(fbuf, out_hbm.at[pl.ds(c * N_PAD + off, AFB)])
            return cy

        lax.fori_loop(0, NPT // AFB, fl, 0)

    return k(y, gidx, dst, w)


BN = 400
NB = N // BN


def _dense(x, acc, rw, rtw, rtb):
    with_acc = acc is not None

    def body(*refs):
        if with_acc:
            x_ref, acc_ref, rw_ref, rtw_ref, rtb_ref, y_ref, root_ref = refs
            h = jnp.maximum(x_ref[...] + acc_ref[0] + acc_ref[1], 0.0)
        else:
            x_ref, rw_ref, rtw_ref, rtb_ref, y_ref, root_ref = refs
            h = x_ref[...]
        root_ref[...] = (
            jnp.dot(h, rtw_ref[...], preferred_element_type=jnp.float32)
            + rtb_ref[...]
        )
        for r in range(R):
            y_ref[r] = jnp.dot(h, rw_ref[r], preferred_element_type=jnp.float32)

    in_specs = [pl.BlockSpec((BN, D), lambda i: (i, 0))]
    operands = [x]
    if with_acc:
        in_specs.append(pl.BlockSpec((NC, BN, D), lambda i: (0, i, 0)))
        operands.append(acc)
    in_specs += [
        pl.BlockSpec((R, D, D), lambda i: (0, 0, 0)),
        pl.BlockSpec((D, D), lambda i: (0, 0)),
        pl.BlockSpec((1, D), lambda i: (0, 0)),
    ]
    operands += [rw, rtw, rtb]
    return pl.pallas_call(
        body,
        grid=(NB,),
        in_specs=in_specs,
        out_specs=[
            pl.BlockSpec((R, BN, D), lambda i: (0, i, 0)),
            pl.BlockSpec((BN, D), lambda i: (i, 0)),
        ],
        out_shape=[
            jax.ShapeDtypeStruct((R, N, D), jnp.float32),
            jax.ShapeDtypeStruct((N, D), jnp.float32),
        ],
    )(*operands)


def _head(root, acc, batch2, w1, b1, w2, b2, w3, b3):
    def body(root_ref, acc_ref, b_ref, w1_ref, b1_ref, w2_ref, b2_ref, w3_ref,
             b3_ref, logits_ref, sums_ref, cnts_ref):
        i = pl.program_id(0)
        h = jnp.maximum(root_ref[...] + acc_ref[0] + acc_ref[1], 0.0)
        gids = lax.broadcasted_iota(jnp.int32, (1, G), 1)
        onehot = (b_ref[...] == gids).astype(jnp.float32)
        dnum = (((0,), (0,)), ((), ()))
        ssum = lax.dot_general(onehot, h, dnum,
                               preferred_element_type=jnp.float32)
        scnt = lax.dot_general(onehot, jnp.ones((BN, 1), jnp.float32), dnum,
                               preferred_element_type=jnp.float32)

        @pl.when(i == 0)
        def _():
            sums_ref[...] = ssum
            cnts_ref[...] = scnt

        @pl.when(i > 0)
        def _():
            sums_ref[...] += ssum
            cnts_ref[...] += scnt

        @pl.when(i == NB - 1)
        def _():
            pooled = sums_ref[...] / jnp.maximum(cnts_ref[...], 1.0)
            z = jnp.maximum(
                jnp.dot(pooled, w1_ref[...],
                        preferred_element_type=jnp.float32) + b1_ref[...], 0.0)
            z = jnp.maximum(
                jnp.dot(z, w2_ref[...],
                        preferred_element_type=jnp.float32) + b2_ref[...], 0.0)
            logits_ref[...] = (
                jnp.dot(z, w3_ref[...], preferred_element_type=jnp.float32)
                + b3_ref[...])

    logits, _, _ = pl.pallas_call(
        body,
        grid=(NB,),
        in_specs=[
            pl.BlockSpec((BN, D), lambda i: (i, 0)),
            pl.BlockSpec((NC, BN, D), lambda i: (0, i, 0)),
            pl.BlockSpec((BN, 1), lambda i: (i, 0)),
            pl.BlockSpec((D, D), lambda i: (0, 0)),
            pl.BlockSpec((1, D), lambda i: (0, 0)),
            pl.BlockSpec((D, D), lambda i: (0, 0)),
            pl.BlockSpec((1, D), lambda i: (0, 0)),
            pl.BlockSpec((D, C), lambda i: (0, 0)),
            pl.BlockSpec((1, C), lambda i: (0, 0)),
        ],
        out_specs=[
            pl.BlockSpec((G, C), lambda i: (0, 0)),
            pl.BlockSpec((G, D), lambda i: (0, 0)),
            pl.BlockSpec((G, 1), lambda i: (0, 0)),
        ],
        out_shape=[
            jax.ShapeDtypeStruct((G, C), jnp.float32),
            jax.ShapeDtypeStruct((G, D), jnp.float32),
            jax.ShapeDtypeStruct((G, 1), jnp.float32),
        ],
    )(root, acc, batch2, w1, b1, w2, b2, w3, b3)
    return logits


def kernel(x, edge_index, edge_type, batch, rel_w, root_w, root_b,
           mlp_w1, mlp_b1, mlp_w2, mlp_b2, mlp_w3, mlp_b3):
    src = edge_index[0]
    dst = edge_index[1]
    et = edge_type
    seg = dst * R + et
    gidx = et * N + src

    cnt = _count(seg)
    w = _weights(cnt, seg)

    y1, root1 = _dense(x, None, rel_w[0], root_w[0], root_b[0].reshape(1, D))
    acc1 = _agg(y1.reshape(R * N, D), gidx, dst, w)
    acc1 = acc1.reshape(NC, N_PAD, D)[:, :N]
    y2, root2 = _dense(root1, acc1, rel_w[1], root_w[1],
                       root_b[1].reshape(1, D))
    acc2 = _agg(y2.reshape(R * N, D), gidx, dst, w)
    acc2 = acc2.reshape(NC, N_PAD, D)[:, :N]
    logits = _head(root2, acc2, batch.reshape(N, 1),
                   mlp_w1, mlp_b1.reshape(1, D), mlp_w2, mlp_b2.reshape(1, D),
                   mlp_w3, mlp_b3.reshape(1, C))
    return logits
